# baseline (speedup 1.0000x reference)
import functools

import jax
import jax.numpy as jnp
from jax import lax
from jax.experimental import pallas as pl
from jax.experimental.pallas import tpu as pltpu
from jax.experimental.pallas import tpu_sc as plsc

_NC = 2
_NS = 16
_NW = _NC * _NS

_SUB = 128


@functools.partial(jax.jit, static_argnames=("n", "d"))
def _sc_gather(idx_flat, table, n, d):
    v = table.shape[0]
    chunk = 2 * _SUB
    n_per_w = n // _NW
    n_chunks = n_per_w // chunk
    n_pairs = (n_chunks - 1) // 2

    mesh = plsc.VectorSubcoreMesh(core_axis_name="c", subcore_axis_name="s")

    @functools.partial(
        pl.kernel,
        out_type=jax.ShapeDtypeStruct((n, d), jnp.float32),
        mesh=mesh,
        scratch_types=[
            pltpu.VMEM((n_per_w,), jnp.int32),
            pltpu.VMEM_SHARED((v, d), jnp.float32),
            pltpu.VMEM((2 * _SUB, d), jnp.float32),
            pltpu.VMEM((2 * _SUB, d), jnp.float32),
            pltpu.SemaphoreType.DMA,
            pltpu.SemaphoreType.DMA,
            pltpu.SemaphoreType.DMA,
            pltpu.SemaphoreType.DMA,
        ],
    )
    def k(idx_hbm, table_hbm, out_hbm, idx_v, table_v, rows0, rows1,
          g0, g1, w0, w1):
        wid = lax.axis_index("s") * _NC + lax.axis_index("c")
        base = wid * n_per_w

        pltpu.async_copy(idx_hbm.at[pl.ds(base, n_per_w)], idx_v, g0)

        @pl.when(lax.axis_index("s") == 0)
        def _():
            pltpu.sync_copy(table_hbm, table_v)

        plsc.subcore_barrier()
        pltpu.make_async_copy(idx_hbm.at[pl.ds(0, n_per_w)], idx_v, g0).wait()

        def g0_start(c):
            pltpu.async_copy(table_v.at[idx_v.at[pl.ds(c * chunk, _SUB)]],
                             rows0.at[pl.ds(0, _SUB)], g0)
            pltpu.async_copy(
                table_v.at[idx_v.at[pl.ds(c * chunk + _SUB, _SUB)]],
                rows0.at[pl.ds(_SUB, _SUB)], g0)

        def g0_wait():
            pltpu.make_async_copy(table_v.at[idx_v.at[pl.ds(0, _SUB)]],
                                  rows0.at[pl.ds(0, _SUB)], g0).wait()
            pltpu.make_async_copy(table_v.at[idx_v.at[pl.ds(0, _SUB)]],
                                  rows0.at[pl.ds(_SUB, _SUB)], g0).wait()

        def g1_start(c):
            pltpu.async_copy(table_v.at[idx_v.at[pl.ds(c * chunk, _SUB)]],
                             rows1.at[pl.ds(0, _SUB)], g1)
            pltpu.async_copy(
                table_v.at[idx_v.at[pl.ds(c * chunk + _SUB, _SUB)]],
                rows1.at[pl.ds(_SUB, _SUB)], g1)

        def g1_wait():
            pltpu.make_async_copy(table_v.at[idx_v.at[pl.ds(0, _SUB)]],
                                  rows1.at[pl.ds(0, _SUB)], g1).wait()
            pltpu.make_async_copy(table_v.at[idx_v.at[pl.ds(0, _SUB)]],
                                  rows1.at[pl.ds(_SUB, _SUB)], g1).wait()

        def w0_start(c):
            pltpu.async_copy(rows0, out_hbm.at[pl.ds(base + c * chunk, chunk)],
                             w0)

        def w0_wait():
            pltpu.make_async_copy(rows0, out_hbm.at[pl.ds(base, chunk)],
                                  w0).wait()

        def w1_start(c):
            pltpu.async_copy(rows1, out_hbm.at[pl.ds(base + c * chunk, chunk)],
                             w1)

        def w1_wait():
            pltpu.make_async_copy(rows1, out_hbm.at[pl.ds(base, chunk)],
                                  w1).wait()

        g0_start(0)

        def pair(r, carry):
            @pl.when(r > 0)
            def _():
                w1_wait()

            g1_start(2 * r + 1)
            g0_wait()
            w0_start(2 * r)

            w0_wait()
            g0_start(2 * r + 2)

            g1_wait()
            w1_start(2 * r + 1)
            return carry

        lax.fori_loop(0, n_pairs, pair, 0)

        g0_wait()
        w0_start(n_chunks - 1)
        w1_wait()
        w0_wait()

    return k(idx_flat, table)


def kernel(actions, embed_weight):
    b, a, l = actions.shape
    v, d = embed_weight.shape
    n = b * a * l
    out = _sc_gather(actions.reshape(n), embed_weight, n, d)
    return out.reshape(b, a * l, d)


# speedup vs baseline: 1.0114x; 1.0114x over previous
import functools

import jax
import jax.numpy as jnp
from jax import lax
from jax.experimental import pallas as pl
from jax.experimental.pallas import tpu as pltpu
from jax.experimental.pallas import tpu_sc as plsc

_NC = 2
_NS = 16
_NW = _NC * _NS

_SUB = 128


@functools.partial(jax.jit, static_argnames=("n", "d"))
def _sc_gather(idx_flat, table, n, d):
    v = table.shape[0]
    n_per_w = n // _NW
    n_chunks = n_per_w // _SUB
    n_pairs = n_chunks // 2

    mesh = plsc.VectorSubcoreMesh(core_axis_name="c", subcore_axis_name="s")

    @functools.partial(
        pl.kernel,
        out_type=jax.ShapeDtypeStruct((n, d), jnp.float32),
        mesh=mesh,
        scratch_types=[
            pltpu.VMEM((n_per_w,), jnp.int32),
            pltpu.VMEM_SHARED((v, d), jnp.float32),
            pltpu.VMEM((_SUB, d), jnp.float32),
            pltpu.VMEM((_SUB, d), jnp.float32),
            pltpu.SemaphoreType.DMA,
            pltpu.SemaphoreType.DMA,
            pltpu.SemaphoreType.DMA,
            pltpu.SemaphoreType.DMA,
        ],
    )
    def k(idx_hbm, table_hbm, out_hbm, idx_v, table_v, rows0, rows1,
          g0, g1, w0, w1):
        wid = lax.axis_index("s") * _NC + lax.axis_index("c")
        base = wid * n_per_w

        pltpu.async_copy(idx_hbm.at[pl.ds(base, n_per_w)], idx_v, g0)

        @pl.when(lax.axis_index("s") == 0)
        def _():
            pltpu.sync_copy(table_hbm, table_v)

        plsc.subcore_barrier()
        pltpu.make_async_copy(idx_hbm.at[pl.ds(0, n_per_w)], idx_v, g0).wait()

        def g0_start(c):
            pltpu.async_copy(table_v.at[idx_v.at[pl.ds(c * _SUB, _SUB)]],
                             rows0, g0)

        def g0_wait():
            pltpu.make_async_copy(table_v.at[idx_v.at[pl.ds(0, _SUB)]],
                                  rows0, g0).wait()

        def g1_start(c):
            pltpu.async_copy(table_v.at[idx_v.at[pl.ds(c * _SUB, _SUB)]],
                             rows1, g1)

        def g1_wait():
            pltpu.make_async_copy(table_v.at[idx_v.at[pl.ds(0, _SUB)]],
                                  rows1, g1).wait()

        def w0_start(c):
            pltpu.async_copy(rows0, out_hbm.at[pl.ds(base + c * _SUB, _SUB)],
                             w0)

        def w0_wait():
            pltpu.make_async_copy(rows0, out_hbm.at[pl.ds(base, _SUB)],
                                  w0).wait()

        def w1_start(c):
            pltpu.async_copy(rows1, out_hbm.at[pl.ds(base + c * _SUB, _SUB)],
                             w1)

        def w1_wait():
            pltpu.make_async_copy(rows1, out_hbm.at[pl.ds(base, _SUB)],
                                  w1).wait()

        g0_start(0)

        def pair(r, carry):
            @pl.when(r > 0)
            def _():
                w1_wait()

            g1_start(2 * r + 1)
            g0_wait()
            w0_start(2 * r)

            w0_wait()

            @pl.when(r + 1 < n_pairs)
            def _():
                g0_start(2 * r + 2)

            g1_wait()
            w1_start(2 * r + 1)
            return carry

        lax.fori_loop(0, n_pairs, pair, 0)
        w1_wait()

    return k(idx_flat, table)


def kernel(actions, embed_weight):
    b, a, l = actions.shape
    v, d = embed_weight.shape
    n = b * a * l
    out = _sc_gather(actions.reshape(n), embed_weight, n, d)
    return out.reshape(b, a * l, d)
